# baseline (speedup 1.0000x reference)
import functools

import jax
import jax.numpy as jnp
from jax import lax
from jax.experimental import pallas as pl
from jax.experimental.pallas import tpu as pltpu
from jax.experimental.pallas import tpu_sc as plsc

K = 1000000
OUT_F = 16384
IN_F = 64
B = OUT_F * IN_F

NC = 2
NS = 16
NW = NC * NS
HROWS = IN_F // 2
HB = OUT_F * HROWS

_MESH = plsc.VectorSubcoreMesh(
    core_axis_name="c", subcore_axis_name="s", num_cores=NC, num_subcores=NS
)


def _make_half_gather(row0):
    @functools.partial(
        pl.kernel,
        out_type=[
            jax.ShapeDtypeStruct((HROWS, OUT_F), jnp.float32),
            jax.ShapeDtypeStruct((HROWS, OUT_F), jnp.float32),
        ],
        mesh=_MESH,
        scratch_types=[
            pltpu.VMEM((OUT_F,), jnp.int32),
            pltpu.VMEM((OUT_F,), jnp.float32),
            pltpu.VMEM((OUT_F,), jnp.float32),
            pltpu.SemaphoreType.DMA,
            pltpu.SemaphoreType.DMA,
        ],
    )
    def _half(mu_hbm, rho_hbm, idx_hbm, mug_hbm, rhog_hbm,
              idx_v, mug_v, rhog_v, sem_mu, sem_rho):
        wid = lax.axis_index("s") * NC + lax.axis_index("c")
        pltpu.sync_copy(idx_hbm.at[pl.ds((row0 + wid) * OUT_F, OUT_F)], idx_v)
        cp_mu = pltpu.async_copy(mu_hbm.at[0].at[idx_v], mug_v, sem_mu)
        cp_rho = pltpu.async_copy(rho_hbm.at[0].at[idx_v], rhog_v, sem_rho)
        cp_mu.wait()
        pltpu.sync_copy(mug_v, mug_hbm.at[wid, pl.ds(0, OUT_F)])
        cp_rho.wait()
        pltpu.sync_copy(rhog_v, rhog_hbm.at[wid, pl.ds(0, OUT_F)])

    return _half


_sc_gather_a = _make_half_gather(0)
_sc_gather_b = _make_half_gather(HROWS)

_BLK = 2048


def _tc_finish_a_body(mu_ref, rho_ref, eps_ref, out_ref):
    sigma = jnp.log1p(jnp.exp(rho_ref[...]))
    out_ref[...] = mu_ref[...] + sigma * eps_ref[...]


_tc_finish_a = pl.pallas_call(
    _tc_finish_a_body,
    grid=(OUT_F // _BLK,),
    in_specs=[
        pl.BlockSpec((HROWS, _BLK), lambda i: (0, i)),
        pl.BlockSpec((HROWS, _BLK), lambda i: (0, i)),
        pl.BlockSpec((HROWS, _BLK), lambda i: (0, i)),
    ],
    out_specs=pl.BlockSpec((HROWS, _BLK), lambda i: (0, i)),
    out_shape=jax.ShapeDtypeStruct((IN_F, OUT_F), jnp.float32),
)


def _tc_finish_b_body(mu_ref, rho_ref, eps_ref, _prev_ref, out_ref):
    sigma = jnp.log1p(jnp.exp(rho_ref[...]))
    out_ref[...] = mu_ref[...] + sigma * eps_ref[...]


_tc_finish_b = pl.pallas_call(
    _tc_finish_b_body,
    grid=(OUT_F // _BLK,),
    in_specs=[
        pl.BlockSpec((HROWS, _BLK), lambda i: (0, i)),
        pl.BlockSpec((HROWS, _BLK), lambda i: (0, i)),
        pl.BlockSpec((HROWS, _BLK), lambda i: (1, i)),
        pl.BlockSpec(memory_space=pltpu.MemorySpace.HBM),
    ],
    out_specs=pl.BlockSpec((HROWS, _BLK), lambda i: (1, i)),
    out_shape=jax.ShapeDtypeStruct((IN_F, OUT_F), jnp.float32),
    input_output_aliases={3: 0},
)


def kernel(weight_mu_share, weight_rho_share, eps_w, indices):
    idx_t = jnp.transpose(indices[0], (1, 0)).reshape(B)
    eps_t = jnp.transpose(eps_w[0], (1, 0))
    mu_a, rho_a = _sc_gather_a(weight_mu_share, weight_rho_share, idx_t)
    mu_b, rho_b = _sc_gather_b(weight_mu_share, weight_rho_share, idx_t)
    out_a = _tc_finish_a(mu_a, rho_a, eps_t)
    return _tc_finish_b(mu_b, rho_b, eps_t, out_a)
